# baseline (speedup 1.0000x reference)
import functools
import math

import jax
import jax.numpy as jnp
import numpy as np
from jax import lax
from jax.experimental import pallas as pl
from jax.experimental.pallas import tpu as pltpu
from jax.experimental.pallas import tpu_sc as plsc

_NC = 2
_NS = 16
_NW = _NC * _NS

_IDXROW = 128
_GPC = 4
_ROWS = _IDXROW * _GPC
_LANES = 16


@functools.lru_cache(maxsize=None)
def _build_call(n_rows: int, hidden: int, scale: float):
    assert n_rows % (_NW * _ROWS) == 0
    n_idxrows = n_rows // _IDXROW
    rows_per_worker = n_idxrows // _NW
    chunks = rows_per_worker // _GPC
    vecs_per_row = hidden // _LANES

    mesh = plsc.VectorSubcoreMesh(core_axis_name="c", subcore_axis_name="s")

    @functools.partial(
        pl.kernel,
        mesh=mesh,
        out_type=jax.ShapeDtypeStruct((n_rows, hidden), jnp.float32),
        scratch_types=[
            pltpu.VMEM((_GPC, _IDXROW), jnp.int32),
            pltpu.VMEM((_ROWS, hidden), jnp.float32),
            pltpu.SemaphoreType.DMA,
        ],
        compiler_params=pltpu.CompilerParams(use_tc_tiling_on_sc=False),
    )
    def emb(idx_hbm, table_hbm, out_hbm, idx_v, rows_v, gsem):
        wid = lax.axis_index("s") * _NC + lax.axis_index("c")
        row0 = wid * rows_per_worker

        def chunk_body(g, carry):
            ib = row0 + g * _GPC
            pltpu.sync_copy(idx_hbm.at[pl.ds(ib, _GPC)], idx_v)
            descs = []
            for j in range(_GPC):
                descs.append(pltpu.async_copy(
                    table_hbm.at[idx_v.at[j]],
                    rows_v.at[pl.ds(j * _IDXROW, _IDXROW)],
                    gsem,
                ))
            for d in descs:
                d.wait()

            def scale_body(i, c):
                for jj in range(vecs_per_row):
                    sl = pl.ds(jj * _LANES, _LANES)
                    rows_v[i, sl] = rows_v[i, sl] * scale
                return c

            lax.fori_loop(0, _ROWS, scale_body, 0, unroll=4)

            pltpu.sync_copy(rows_v, out_hbm.at[pl.ds(ib * _IDXROW, _ROWS)])
            return carry

        lax.fori_loop(0, chunks, chunk_body, 0)

    return emb


def kernel(x, weights):
    n_rows = x.shape[0] * x.shape[1]
    hidden = weights.shape[1]
    scale = float(np.float32(np.sqrt(np.float32(hidden))))
    idx2d = x.reshape(n_rows // _IDXROW, _IDXROW).astype(jnp.int32)
    out = _build_call(n_rows, hidden, scale)(idx2d, weights)
    return out.reshape(x.shape[0], x.shape[1], hidden)
